# Initial kernel scaffold; baseline (speedup 1.0000x reference)
import jax
import jax.numpy as jnp
from jax.experimental import pallas as pl


def kernel(src_feat, mp1_feat1, mp1_feat2, mp2_feat1, mp2_feat2, mp1_idx0, mp1_idx1, mp1_idx2, mp2_idx0, mp2_idx1, mp2_idx2, W_src, b_src, W_B, b_B, mp1_featW, mp1_featb, mp1_attW, mp1_attb, mp1_bias, mp2_featW, mp2_featb, mp2_attW, mp2_attb, mp2_bias, sem_W, sem_b, res_inW, res_inb, res_Ws, res_bs, res_outW, res_outb):
    raise NotImplementedError("write your pallas kernel here")


# speedup vs baseline: 12.8643x; 12.8643x over previous
import functools

import jax
import jax.numpy as jnp
from jax import lax
from jax.experimental import pallas as pl
from jax.experimental.pallas import tpu as pltpu
from jax.experimental.pallas import tpu_sc as plsc

N_SRC = 10000
N_B = 10000
E = 320000
D_IN = 128
D = 64
DP = 80
NT = 16
EPT = E // NT
BLK = 80
NBLK = 25
CHUNK = BLK * NBLK
NCH = EPT // CHUNK
N_PAD = 10240
RPT = N_PAD // NT
ZR = 128
RB = 400
GRID = N_SRC // RB


def _prologue_body(f_ref, w7_ref, b7_ref, sw_ref, sc_ref,
                   srch_ref, t0s_ref, tbl1_ref, tbl2_ref, s8_ref):
    fmap = (0, 0, 1, 2, 0, 3, 4)
    t = []
    for k in range(7):
        t.append(jnp.dot(f_ref[fmap[k]], w7_ref[k],
                         preferred_element_type=jnp.float32) + b7_ref[k][None, :])
    srch_ref[...] = t[0]
    t0s_ref[0] = t[1]
    t0s_ref[1] = t[4]
    tbl1_ref[0] = t[2]
    tbl1_ref[1] = t[5]
    tbl2_ref[0] = t[3]
    tbl2_ref[1] = t[6]
    sfmap = (0, 1, 2, 0, 3, 4)
    cols = []
    for k in range(6):
        cols.append(jnp.sum(f_ref[sfmap[k]] * sw_ref[k][None, :], axis=1)
                    + sc_ref[0, k])
    z = jnp.zeros_like(cols[0])
    s8_ref[...] = jnp.stack(cols + [z, z], axis=1)


def _prologue(F, W7, b7, SW, SC8):
    return pl.pallas_call(
        _prologue_body,
        grid=(GRID,),
        in_specs=[
            pl.BlockSpec((5, RB, D_IN), lambda i: (0, i, 0)),
            pl.BlockSpec((7, D_IN, D), lambda i: (0, 0, 0)),
            pl.BlockSpec((7, D), lambda i: (0, 0)),
            pl.BlockSpec((6, D_IN), lambda i: (0, 0)),
            pl.BlockSpec((1, 8), lambda i: (0, 0)),
        ],
        out_specs=[
            pl.BlockSpec((RB, D), lambda i: (i, 0)),
            pl.BlockSpec((2, RB, D), lambda i: (0, i, 0)),
            pl.BlockSpec((2, RB, D), lambda i: (0, i, 0)),
            pl.BlockSpec((2, RB, D), lambda i: (0, i, 0)),
            pl.BlockSpec((RB, 8), lambda i: (i, 0)),
        ],
        out_shape=[
            jax.ShapeDtypeStruct((N_SRC, D), jnp.float32),
            jax.ShapeDtypeStruct((2, N_SRC, D), jnp.float32),
            jax.ShapeDtypeStruct((2, N_B, D), jnp.float32),
            jax.ShapeDtypeStruct((2, N_B, D), jnp.float32),
            jax.ShapeDtypeStruct((N_SRC, 8), jnp.float32),
        ],
    )(F, W7, b7, SW, SC8)


_MESH = plsc.VectorSubcoreMesh(core_axis_name="c", subcore_axis_name="s")


@functools.partial(
    pl.kernel,
    mesh=_MESH,
    compiler_params=pltpu.CompilerParams(needs_layout_passes=False,
                                         use_tc_tiling_on_sc=False),
    out_type=jax.ShapeDtypeStruct((2, NT, RPT, DP), jnp.float32),
    scratch_types=[
        pltpu.VMEM((N_SRC,), jnp.float32),
        pltpu.VMEM((N_SRC,), jnp.float32),
        pltpu.VMEM((N_SRC,), jnp.float32),
        pltpu.VMEM((NBLK, 1, BLK), jnp.int32),
        pltpu.VMEM((NBLK, 1, BLK), jnp.int32),
        pltpu.VMEM((NBLK, 1, BLK), jnp.int32),
        pltpu.VMEM((BLK, D), jnp.float32),
        pltpu.VMEM((BLK, D), jnp.float32),
        pltpu.VMEM((BLK, DP), jnp.float32),
        pltpu.VMEM((BLK,), jnp.float32),
        pltpu.VMEM((ZR, DP), jnp.float32),
        pltpu.VMEM_SHARED((N_PAD, DP), jnp.float32),
        pltpu.SemaphoreType.DMA,
        pltpu.SemaphoreType.DMA,
    ],
)
def _sc_main(tbl1, tbl2, s0t, s1t, s2t, idx0, idx1, idx2, acc_out,
             s0_v, s1_v, s2_v, i0_v, i1_v, i2_v, r1_v, r2_v, val_v, ex_v,
             zb_v, acc_sh, sem1, sem2):
    c = lax.axis_index("c")
    s = lax.axis_index("s")

    pltpu.sync_copy(s0t.at[c, 0], s0_v)
    pltpu.sync_copy(s1t.at[c, 0], s1_v)
    pltpu.sync_copy(s2t.at[c, 0], s2_v)

    zv = jnp.zeros((16,), jnp.float32)

    def zrow(i, carry):
        for dd in range(DP // 16):
            zb_v[i, pl.ds(dd * 16, 16)] = zv
        return carry

    lax.fori_loop(0, ZR, zrow, 0)
    for k in range(RPT // ZR):
        pltpu.sync_copy(zb_v, acc_sh.at[pl.ds(s * RPT + k * ZR, ZR)])
    plsc.subcore_barrier()

    def chunk_body(ci, carry):
        row0 = s * (NCH * NBLK) + ci * NBLK
        pltpu.sync_copy(idx0.at[c, pl.ds(row0, NBLK)], i0_v)
        pltpu.sync_copy(idx1.at[c, pl.ds(row0, NBLK)], i1_v)
        pltpu.sync_copy(idx2.at[c, pl.ds(row0, NBLK)], i2_v)

        def blk_body(j, carry2):
            h1 = pltpu.async_copy(tbl1.at[i1_v.at[j, 0]], r1_v, sem1)
            h2 = pltpu.async_copy(tbl2.at[i2_v.at[j, 0]], r2_v, sem2)
            for g in range(BLK // 16):
                v0 = i0_v[j, 0, pl.ds(g * 16, 16)]
                v1 = i1_v[j, 0, pl.ds(g * 16, 16)]
                v2 = i2_v[j, 0, pl.ds(g * 16, 16)]
                g0 = plsc.load_gather(s0_v, [v0])
                g1 = plsc.load_gather(s1_v, [v1])
                g2 = plsc.load_gather(s2_v, [v2])
                x = g0 + g1 + g2
                e2 = jnp.exp(x + x)
                th = 1.0 - 2.0 / (e2 + 1.0)
                ex_v[pl.ds(g * 16, 16)] = jnp.exp(th)
            h1.wait()
            h2.wait()

            def grp(kk, carry3):
                exvec = ex_v[pl.ds(kk * 16, 16)]
                for e in range(16):
                    ee = kk * 16 + e
                    b = jnp.full((16,), exvec[e], jnp.float32)
                    for dd in range(D // 16):
                        row = (r1_v[ee, pl.ds(dd * 16, 16)]
                               + r2_v[ee, pl.ds(dd * 16, 16)])
                        val_v[ee, pl.ds(dd * 16, 16)] = b * row
                    val_v[ee, pl.ds(D, 16)] = b
                return carry3

            lax.fori_loop(0, BLK // 16, grp, 0)
            pltpu.sync_copy(val_v, acc_sh.at[i0_v.at[j, 0]], add=True)
            return carry2

        lax.fori_loop(0, NBLK, blk_body, 0)
        return carry

    lax.fori_loop(0, NCH, chunk_body, 0)
    plsc.subcore_barrier()
    pltpu.sync_copy(acc_sh.at[pl.ds(s * RPT, RPT)], acc_out.at[c, s])


def _epilogue_body(srch_ref, t0s_ref, acc_ref, mpb_ref, semv_ref,
                   riw_ref, rib_ref, rws_ref, rbs_ref, rov_ref, scal_ref,
                   out_ref):
    hs = [srch_ref[...]]
    for cc in range(2):
        a = acc_ref[cc]
        den = a[:, D:D + 1]
        ind = (den > 0).astype(jnp.float32)
        hp = (t0s_ref[cc] * ind + a[:, 0:D] / jnp.maximum(den, 1e-30)) / 3.0
        hs.append(hp + mpb_ref[cc][None, :])
    semb = scal_ref[0, 0]
    atts = [jnp.sum(h * semv_ref[0][None, :], axis=1, keepdims=True) + semb
            for h in hs]
    att = jnp.concatenate(atts, axis=1)
    att = jnp.where(att > 0, att, 0.01 * att)
    m = jnp.max(att, axis=1, keepdims=True)
    exa = jnp.exp(att - m)
    w = exa / jnp.sum(exa, axis=1, keepdims=True)
    hp = w[:, 0:1] * hs[0] + w[:, 1:2] * hs[1] + w[:, 2:3] * hs[2]
    h = jnp.maximum(jnp.dot(hp, riw_ref[...],
                            preferred_element_type=jnp.float32)
                    + rib_ref[0][None, :], 0.0)
    i = 0
    for _blk in range(2):
        r = h
        for _l in range(2):
            h = jnp.maximum(jnp.dot(h, rws_ref[i],
                                    preferred_element_type=jnp.float32)
                            + rbs_ref[i][None, :], 0.0)
            i += 1
        h = h + r
    o = jnp.sum(h * rov_ref[0][None, :], axis=1, keepdims=True) + scal_ref[0, 1]
    out_ref[...] = 1.0 / (1.0 + jnp.exp(-o))


def _epilogue(srch, t0s, ACC, mpb, semv, riw, rib, rws, rbs, rov, scal):
    return pl.pallas_call(
        _epilogue_body,
        grid=(GRID,),
        in_specs=[
            pl.BlockSpec((RB, D), lambda i: (i, 0)),
            pl.BlockSpec((2, RB, D), lambda i: (0, i, 0)),
            pl.BlockSpec((2, RB, DP), lambda i: (0, i, 0)),
            pl.BlockSpec((2, D), lambda i: (0, 0)),
            pl.BlockSpec((1, D), lambda i: (0, 0)),
            pl.BlockSpec((D, D), lambda i: (0, 0)),
            pl.BlockSpec((1, D), lambda i: (0, 0)),
            pl.BlockSpec((4, D, D), lambda i: (0, 0, 0)),
            pl.BlockSpec((4, D), lambda i: (0, 0)),
            pl.BlockSpec((1, D), lambda i: (0, 0)),
            pl.BlockSpec((1, 8), lambda i: (0, 0)),
        ],
        out_specs=pl.BlockSpec((RB, 1), lambda i: (i, 0)),
        out_shape=jax.ShapeDtypeStruct((N_SRC, 1), jnp.float32),
    )(srch, t0s, ACC, mpb, semv, riw, rib, rws, rbs, rov, scal)


def kernel(src_feat, mp1_feat1, mp1_feat2, mp2_feat1, mp2_feat2,
           mp1_idx0, mp1_idx1, mp1_idx2, mp2_idx0, mp2_idx1, mp2_idx2,
           W_src, b_src, W_B, b_B,
           mp1_featW, mp1_featb, mp1_attW, mp1_attb, mp1_bias,
           mp2_featW, mp2_featb, mp2_attW, mp2_attb, mp2_bias,
           sem_W, sem_b, res_inW, res_inb, res_Ws, res_bs, res_outW, res_outb):
    f32 = jnp.float32
    w7, b7, sw, sc = [W_src], [b_src], [], []
    for fW, fb, attW, attb in ((mp1_featW, mp1_featb, mp1_attW, mp1_attb),
                               (mp2_featW, mp2_featb, mp2_attW, mp2_attb)):
        w0 = W_src @ fW
        wb = W_B @ fW
        b0 = b_src @ fW + fb
        bb = b_B @ fW + fb
        w7 += [w0, wb, wb]
        b7 += [b0, bb, bb]
        a1 = attW[:D, 0]
        a2 = attW[D:, 0]
        v0 = a1 + a2 / 3.0
        v12 = a2 / 3.0
        sw += [w0 @ v0, wb @ v12, wb @ v12]
        sc += [jnp.dot(b0, v0) + attb[0], jnp.dot(bb, v12), jnp.dot(bb, v12)]
    W7 = jnp.stack(w7).astype(f32)
    B7 = jnp.stack(b7).astype(f32)
    SW = jnp.stack(sw).astype(f32)
    SC8 = jnp.concatenate([jnp.stack(sc), jnp.zeros((2,), f32)]).reshape(1, 8)

    F = jnp.stack([src_feat, mp1_feat1, mp1_feat2, mp2_feat1, mp2_feat2])
    srch, t0s, tbl1_3, tbl2_3, s8 = _prologue(F, W7, B7, SW, SC8)
    tbl1 = tbl1_3.reshape(2 * N_B, D)
    tbl2 = tbl2_3.reshape(2 * N_B, D)
    sT = s8.T
    s0t = jnp.stack([sT[0], sT[3]]).reshape(2, 1, N_SRC)
    s1t = jnp.stack([sT[1], sT[4]]).reshape(2, 1, N_SRC)
    s2t = jnp.stack([sT[2], sT[5]]).reshape(2, 1, N_SRC)

    i32 = jnp.int32
    IDX0 = jnp.stack([mp1_idx0, mp2_idx0]).astype(i32).reshape(2, E // BLK, 1, BLK)
    IDX1 = jnp.stack([mp1_idx1, mp2_idx1 + N_B]).astype(i32).reshape(2, E // BLK, 1, BLK)
    IDX2 = jnp.stack([mp1_idx2, mp2_idx2 + N_B]).astype(i32).reshape(2, E // BLK, 1, BLK)

    ACCp = _sc_main(tbl1, tbl2, s0t, s1t, s2t, IDX0, IDX1, IDX2)
    ACC = ACCp.reshape(2, N_PAD, DP)[:, :N_SRC]

    mpb = jnp.stack([mp1_bias, mp2_bias]).astype(f32)
    semv = sem_W[:, 0].reshape(1, D)
    rov = res_outW[:, 0].reshape(1, D)
    rib = res_inb.reshape(1, D)
    scal = jnp.concatenate([sem_b, res_outb, jnp.zeros((6,), f32)]).reshape(1, 8)
    return _epilogue(srch, t0s, ACC, mpb, semv, res_inW, rib,
                     res_Ws, res_bs, rov, scal)
